# baseline (speedup 1.0000x reference)
import functools

import jax
import jax.numpy as jnp
from jax import lax
from jax.experimental import pallas as pl
from jax.experimental.pallas import tpu as pltpu
from jax.experimental.pallas import tpu_sc as plsc

N = 10000
E = 320000
F_IN = 128
H = 256
C = 40

NW = 32
R = 320
NPAD = NW * R
CH = 2560
NCH = E // CH
NEG_INF = float("-inf")


def _make_segmax(F, G=32):
    mesh = plsc.VectorSubcoreMesh(
        core_axis_name="c", subcore_axis_name="s", num_cores=2, num_subcores=16
    )

    @functools.partial(
        pl.kernel,
        mesh=mesh,
        out_type=jax.ShapeDtypeStruct((NPAD, F), jnp.float32),
        compiler_params=pltpu.CompilerParams(needs_layout_passes=False),
        scratch_types=[
            pltpu.VMEM((2, 2, CH), jnp.int32),
            pltpu.VMEM((CH,), jnp.int32),
            pltpu.VMEM((CH,), jnp.int32),
            pltpu.VMEM((2, G, F), jnp.float32),
            pltpu.VMEM((R + 1, F), jnp.float32),
            pltpu.SemaphoreType.DMA,
            pltpu.SemaphoreType.DMA,
            pltpu.SemaphoreType.DMA,
            pltpu.SemaphoreType.DMA,
        ],
    )
    def segmax(ei_hbm, table_hbm, out_hbm, stage, pend_src, pend_dst, rows,
               acc, sem_s0, sem_s1, sem_g0, sem_g1):
        wid = lax.axis_index("s") * 2 + lax.axis_index("c")
        lo = wid * R
        sem_s = (sem_s0, sem_s1)
        sem_g = (sem_g0, sem_g1)

        def init_acc(r, _):
            for j in range(F // 16):
                acc[r, pl.ds(16 * j, 16)] = jnp.full((16,), NEG_INF, jnp.float32)
            return 0

        lax.fori_loop(0, R + 1, init_acc, 0)

        def init_pend(i, _):
            pend_src[pl.ds(16 * i, 16)] = jnp.zeros((16,), jnp.int32)
            return 0

        lax.fori_loop(0, CH // 16, init_pend, 0)

        def fire_stage(c, b):
            pltpu.async_copy(
                ei_hbm.at[:, pl.ds(c * CH, CH)], stage.at[b], sem_s[b]
            )

        def wait_stage(c, b):
            pltpu.make_async_copy(
                ei_hbm.at[:, pl.ds(c * CH, CH)], stage.at[b], sem_s[b]
            ).wait()

        def fire_gather(g, q):
            pltpu.async_copy(
                table_hbm.at[pend_src.at[pl.ds(g * G, G)]], rows.at[q], sem_g[q]
            )

        def wait_gather(g, q):
            pltpu.make_async_copy(
                table_hbm.at[pend_src.at[pl.ds(g * G, G)]], rows.at[q], sem_g[q]
            ).wait()

        def process_group(g, q, cnt):
            for k in range(G // 16):
                dv = pend_dst[pl.ds(g * G + 16 * k, 16)] - lo
                evec = g * G + 16 * k + lax.iota(jnp.int32, 16)
                dsafe = jnp.where(evec < cnt, dv, R)
                dsc = [dsafe[i] for i in range(16)]

                def jblk(jb, _):
                    base = 64 * jb
                    for i in range(16):
                        d = dsc[i]
                        row = [
                            rows[q, 16 * k + i, pl.ds(base + 16 * j, 16)]
                            for j in range(4)
                        ]
                        cur = [
                            acc[d, pl.ds(base + 16 * j, 16)] for j in range(4)
                        ]
                        for j in range(4):
                            acc[d, pl.ds(base + 16 * j, 16)] = jnp.maximum(
                                cur[j], row[j]
                            )
                    return 0

                lax.fori_loop(0, F // 64, jblk, 0)

        def do_chunk(c, b):
            wait_stage(c, b)

            @pl.when(c + 1 < NCH)
            def _():
                fire_stage(c + 1, 1 - b)

            def filt(s, cnt):
                sv = stage[b, 0, pl.ds(16 * s, 16)]
                dv = stage[b, 1, pl.ds(16 * s, 16)]
                m = (dv >= lo) & (dv < lo + R)
                cum = jnp.cumsum(jnp.where(m, 1, 0))
                pos = cnt + cum - 1
                plsc.store_scatter(pend_src, [pos], sv, mask=m)
                plsc.store_scatter(pend_dst, [pos], dv, mask=m)
                return cnt + cum[15]

            cnt = lax.fori_loop(0, CH // 16, filt, jnp.int32(0))
            ngroups = (cnt + (G - 1)) // G

            @pl.when(ngroups > 0)
            def _():
                fire_gather(0, 0)

            def gpair(p, _):
                for q in range(2):
                    g = 2 * p + q

                    @pl.when(g < ngroups)
                    def _():
                        wait_gather(g, q)

                        @pl.when(g + 1 < ngroups)
                        def _():
                            fire_gather(g + 1, 1 - q)

                        process_group(g, q, cnt)

                return 0

            lax.fori_loop(0, (ngroups + 1) // 2, gpair, 0)

        fire_stage(0, 0)

        def chunk_pair(p, _):
            do_chunk(2 * p, 0)

            @pl.when(2 * p + 1 < NCH)
            def _():
                do_chunk(2 * p + 1, 1)

            return 0

        lax.fori_loop(0, (NCH + 1) // 2, chunk_pair, 0)

        pltpu.sync_copy(acc.at[pl.ds(0, R)], out_hbm.at[pl.ds(lo, R)])

    return segmax


_segmax_l1 = _make_segmax(F_IN, G=16)
_segmax_l2 = _make_segmax(H, G=16)


def _tc1_body(aggr_ref, x_ref, wl_ref, wr_ref, b_ref, h_ref):
    a = aggr_ref[...]
    a = jnp.where(a != NEG_INF, a, 0.0)
    h = jnp.dot(a, wl_ref[...], preferred_element_type=jnp.float32)
    h = h + jnp.dot(x_ref[...], wr_ref[...], preferred_element_type=jnp.float32)
    h = h + b_ref[...]
    h_ref[...] = jnp.maximum(h, 0.0)


def _tc2_body(aggr_ref, h_ref, wl_ref, wr_ref, b_ref, o_ref):
    a = aggr_ref[...]
    a = jnp.where(a != NEG_INF, a, 0.0)
    logits = jnp.dot(a, wl_ref[...], preferred_element_type=jnp.float32)
    logits = logits + jnp.dot(
        h_ref[...], wr_ref[...], preferred_element_type=jnp.float32
    )
    logits = logits + b_ref[...]
    col = lax.broadcasted_iota(jnp.int32, logits.shape, 1)
    logits = jnp.where(col < C, logits, -1e30)
    m = jnp.max(logits, axis=1, keepdims=True)
    ex = jnp.exp(logits - m)
    s = jnp.sum(ex, axis=1, keepdims=True)
    o_ref[...] = logits - m - jnp.log(s)


_BR = 512


def _tc1(aggr, x_pad, W1l, W1r, b1):
    return pl.pallas_call(
        _tc1_body,
        grid=(NPAD // _BR,),
        in_specs=[
            pl.BlockSpec((_BR, F_IN), lambda i: (i, 0)),
            pl.BlockSpec((_BR, F_IN), lambda i: (i, 0)),
            pl.BlockSpec((F_IN, H), lambda i: (0, 0)),
            pl.BlockSpec((F_IN, H), lambda i: (0, 0)),
            pl.BlockSpec((1, H), lambda i: (0, 0)),
        ],
        out_specs=pl.BlockSpec((_BR, H), lambda i: (i, 0)),
        out_shape=jax.ShapeDtypeStruct((NPAD, H), jnp.float32),
    )(aggr, x_pad, W1l, W1r, b1.reshape(1, H))


def _tc2(aggr, h, W2l_pad, W2r_pad, b2_pad):
    return pl.pallas_call(
        _tc2_body,
        grid=(NPAD // _BR,),
        in_specs=[
            pl.BlockSpec((_BR, H), lambda i: (i, 0)),
            pl.BlockSpec((_BR, H), lambda i: (i, 0)),
            pl.BlockSpec((H, 128), lambda i: (0, 0)),
            pl.BlockSpec((H, 128), lambda i: (0, 0)),
            pl.BlockSpec((1, 128), lambda i: (0, 0)),
        ],
        out_specs=pl.BlockSpec((_BR, 128), lambda i: (i, 0)),
        out_shape=jax.ShapeDtypeStruct((NPAD, 128), jnp.float32),
    )(aggr, h, W2l_pad, W2r_pad, b2_pad.reshape(1, 128))


def kernel(x, edge_index, W1l, b1, W1r, W2l, b2, W2r):
    x_pad = jnp.concatenate(
        [x, jnp.zeros((NPAD - N, F_IN), jnp.float32)], axis=0
    )
    aggr1 = _segmax_l1(edge_index, x_pad)
    h = _tc1(aggr1, x_pad, W1l, W1r, b1)
    aggr2 = _segmax_l2(edge_index, h)
    W2l_pad = jnp.concatenate(
        [W2l, jnp.zeros((H, 128 - C), jnp.float32)], axis=1
    )
    W2r_pad = jnp.concatenate(
        [W2r, jnp.zeros((H, 128 - C), jnp.float32)], axis=1
    )
    b2_pad = jnp.concatenate([b2, jnp.zeros((128 - C,), jnp.float32)])
    out_pad = _tc2(aggr2, h, W2l_pad, W2r_pad, b2_pad)
    return out_pad[:N, :C]


# speedup vs baseline: 1.0445x; 1.0445x over previous
import functools

import jax
import jax.numpy as jnp
from jax import lax
from jax.experimental import pallas as pl
from jax.experimental.pallas import tpu as pltpu
from jax.experimental.pallas import tpu_sc as plsc

N = 10000
E = 320000
F_IN = 128
H = 256
C = 40

NW = 32
R = 320
NPAD = NW * R
CH = 2560
NCH = E // CH
NEG_INF = float("-inf")


def _make_segmax(F, G=32):
    mesh = plsc.VectorSubcoreMesh(
        core_axis_name="c", subcore_axis_name="s", num_cores=2, num_subcores=16
    )

    @functools.partial(
        pl.kernel,
        mesh=mesh,
        out_type=jax.ShapeDtypeStruct((NPAD, F), jnp.float32),
        compiler_params=pltpu.CompilerParams(needs_layout_passes=False),
        scratch_types=[
            pltpu.VMEM((2, 2, CH), jnp.int32),
            pltpu.VMEM((CH,), jnp.int32),
            pltpu.VMEM((CH,), jnp.int32),
            pltpu.VMEM((2, G, F), jnp.float32),
            pltpu.VMEM((R + 1, F), jnp.float32),
            pltpu.SemaphoreType.DMA,
            pltpu.SemaphoreType.DMA,
            pltpu.SemaphoreType.DMA,
            pltpu.SemaphoreType.DMA,
        ],
    )
    def segmax(ei_hbm, table_hbm, out_hbm, stage, pend_src, pend_dst, rows,
               acc, sem_s0, sem_s1, sem_g0, sem_g1):
        wid = lax.axis_index("s") * 2 + lax.axis_index("c")
        lo = wid * R
        sem_s = (sem_s0, sem_s1)
        sem_g = (sem_g0, sem_g1)

        def init_acc(r, _):
            for j in range(F // 16):
                acc[r, pl.ds(16 * j, 16)] = jnp.full((16,), NEG_INF, jnp.float32)
            return 0

        lax.fori_loop(0, R + 1, init_acc, 0)

        def init_pend(i, _):
            pend_src[pl.ds(16 * i, 16)] = jnp.zeros((16,), jnp.int32)
            return 0

        lax.fori_loop(0, CH // 16, init_pend, 0)

        def fire_stage(c, b):
            pltpu.async_copy(
                ei_hbm.at[:, pl.ds(c * CH, CH)], stage.at[b], sem_s[b]
            )

        def wait_stage(c, b):
            pltpu.make_async_copy(
                ei_hbm.at[:, pl.ds(c * CH, CH)], stage.at[b], sem_s[b]
            ).wait()

        def fire_gather(g, q):
            pltpu.async_copy(
                table_hbm.at[pend_src.at[pl.ds(g * G, G)]], rows.at[q], sem_g[q]
            )

        def wait_gather(g, q):
            pltpu.make_async_copy(
                table_hbm.at[pend_src.at[pl.ds(g * G, G)]], rows.at[q], sem_g[q]
            ).wait()

        def process_group(g, q, cnt):
            for k in range(G // 16):
                dv = pend_dst[pl.ds(g * G + 16 * k, 16)] - lo
                evec = g * G + 16 * k + lax.iota(jnp.int32, 16)
                dsafe = jnp.where(evec < cnt, dv, R)
                dsc = [dsafe[i] for i in range(16)]

                def jblk(jb, _):
                    base = 64 * jb
                    for i in range(16):
                        d = dsc[i]
                        row = [
                            rows[q, 16 * k + i, pl.ds(base + 16 * j, 16)]
                            for j in range(4)
                        ]
                        cur = [
                            acc[d, pl.ds(base + 16 * j, 16)] for j in range(4)
                        ]
                        for j in range(4):
                            acc[d, pl.ds(base + 16 * j, 16)] = jnp.maximum(
                                cur[j], row[j]
                            )
                    return 0

                lax.fori_loop(0, F // 64, jblk, 0)

        def do_chunk(c, b):
            wait_stage(c, b)

            @pl.when(c + 1 < NCH)
            def _():
                fire_stage(c + 1, 1 - b)

            def filt(s, cnt):
                sv = stage[b, 0, pl.ds(16 * s, 16)]
                dv = stage[b, 1, pl.ds(16 * s, 16)]
                m = (dv >= lo) & (dv < lo + R)
                cum = jnp.cumsum(jnp.where(m, 1, 0))
                pos = cnt + cum - 1
                plsc.store_scatter(pend_src, [pos], sv, mask=m)
                plsc.store_scatter(pend_dst, [pos], dv, mask=m)
                return cnt + cum[15]

            cnt = lax.fori_loop(0, CH // 16, filt, jnp.int32(0))
            ngroups = (cnt + (G - 1)) // G

            @pl.when(ngroups > 0)
            def _():
                fire_gather(0, 0)

            def gpair(p, _):
                for q in range(2):
                    g = 2 * p + q

                    @pl.when(g < ngroups)
                    def _():
                        wait_gather(g, q)

                        @pl.when(g + 1 < ngroups)
                        def _():
                            fire_gather(g + 1, 1 - q)

                        process_group(g, q, cnt)

                return 0

            lax.fori_loop(0, (ngroups + 1) // 2, gpair, 0)

        fire_stage(0, 0)

        def chunk_pair(p, _):
            do_chunk(2 * p, 0)

            @pl.when(2 * p + 1 < NCH)
            def _():
                do_chunk(2 * p + 1, 1)

            return 0

        lax.fori_loop(0, (NCH + 1) // 2, chunk_pair, 0)

        pltpu.sync_copy(acc.at[pl.ds(0, R)], out_hbm.at[pl.ds(lo, R)])

    return segmax


_segmax_l1 = _make_segmax(F_IN, G=32)
_segmax_l2 = _make_segmax(H, G=16)


def _tc1_body(aggr_ref, x_ref, wl_ref, wr_ref, b_ref, h_ref):
    a = aggr_ref[...]
    a = jnp.where(a != NEG_INF, a, 0.0)
    h = jnp.dot(a, wl_ref[...], preferred_element_type=jnp.float32)
    h = h + jnp.dot(x_ref[...], wr_ref[...], preferred_element_type=jnp.float32)
    h = h + b_ref[...]
    h_ref[...] = jnp.maximum(h, 0.0)


def _tc2_body(aggr_ref, h_ref, wl_ref, wr_ref, b_ref, o_ref):
    a = aggr_ref[...]
    a = jnp.where(a != NEG_INF, a, 0.0)
    logits = jnp.dot(a, wl_ref[...], preferred_element_type=jnp.float32)
    logits = logits + jnp.dot(
        h_ref[...], wr_ref[...], preferred_element_type=jnp.float32
    )
    logits = logits + b_ref[...]
    col = lax.broadcasted_iota(jnp.int32, logits.shape, 1)
    logits = jnp.where(col < C, logits, -1e30)
    m = jnp.max(logits, axis=1, keepdims=True)
    ex = jnp.exp(logits - m)
    s = jnp.sum(ex, axis=1, keepdims=True)
    o_ref[...] = logits - m - jnp.log(s)


_BR = 512


def _tc1(aggr, x_pad, W1l, W1r, b1):
    return pl.pallas_call(
        _tc1_body,
        grid=(NPAD // _BR,),
        in_specs=[
            pl.BlockSpec((_BR, F_IN), lambda i: (i, 0)),
            pl.BlockSpec((_BR, F_IN), lambda i: (i, 0)),
            pl.BlockSpec((F_IN, H), lambda i: (0, 0)),
            pl.BlockSpec((F_IN, H), lambda i: (0, 0)),
            pl.BlockSpec((1, H), lambda i: (0, 0)),
        ],
        out_specs=pl.BlockSpec((_BR, H), lambda i: (i, 0)),
        out_shape=jax.ShapeDtypeStruct((NPAD, H), jnp.float32),
    )(aggr, x_pad, W1l, W1r, b1.reshape(1, H))


def _tc2(aggr, h, W2l_pad, W2r_pad, b2_pad):
    return pl.pallas_call(
        _tc2_body,
        grid=(NPAD // _BR,),
        in_specs=[
            pl.BlockSpec((_BR, H), lambda i: (i, 0)),
            pl.BlockSpec((_BR, H), lambda i: (i, 0)),
            pl.BlockSpec((H, 128), lambda i: (0, 0)),
            pl.BlockSpec((H, 128), lambda i: (0, 0)),
            pl.BlockSpec((1, 128), lambda i: (0, 0)),
        ],
        out_specs=pl.BlockSpec((_BR, 128), lambda i: (i, 0)),
        out_shape=jax.ShapeDtypeStruct((NPAD, 128), jnp.float32),
    )(aggr, h, W2l_pad, W2r_pad, b2_pad.reshape(1, 128))


def kernel(x, edge_index, W1l, b1, W1r, W2l, b2, W2r):
    x_pad = jnp.concatenate(
        [x, jnp.zeros((NPAD - N, F_IN), jnp.float32)], axis=0
    )
    aggr1 = _segmax_l1(edge_index, x_pad)
    h = _tc1(aggr1, x_pad, W1l, W1r, b1)
    aggr2 = _segmax_l2(edge_index, h)
    W2l_pad = jnp.concatenate(
        [W2l, jnp.zeros((H, 128 - C), jnp.float32)], axis=1
    )
    W2r_pad = jnp.concatenate(
        [W2r, jnp.zeros((H, 128 - C), jnp.float32)], axis=1
    )
    b2_pad = jnp.concatenate([b2, jnp.zeros((128 - C,), jnp.float32)])
    out_pad = _tc2(aggr2, h, W2l_pad, W2r_pad, b2_pad)
    return out_pad[:N, :C]
